# baseline (speedup 1.0000x reference)
import jax
import jax.numpy as jnp
import numpy as np
from jax.experimental import pallas as pl
from jax.experimental.pallas import tpu as pltpu

B = 64
IN_FEATS = 64
NUM_NODES = 16
D_NODE = 4
HID = 128
H = W = 224
WPAD = 256
HPAD = 232
NB = NUM_NODES * B


def _conv_pool_kernel(x_ref, w_ref, out_ref, xq_ref):
    def mm(a, b):
        return jax.lax.dot_general(a, b, (((1,), (0,)), ((), ())),
                                   preferred_element_type=jnp.float32)

    i = pl.program_id(0)

    @pl.when(i == 0)
    def _zero():
        xq_ref[:] = jnp.zeros((2, 3, 248, 384), jnp.bfloat16)

    @pl.when(i < B)
    def _prep():
        s = jax.lax.rem(i, 2)
        xq_ref[s, :, 16:240, 128:352] = x_ref[0].astype(jnp.bfloat16)

    @pl.when(i > 0)
    def _compute():
        sp = jax.lax.rem(i + 1, 2)
        accs = [jnp.zeros((IN_FEATS, WPAD), jnp.float32) for _ in range(4)]
        ones_row = jnp.ones((1, WPAD), jnp.bfloat16)
        for c in range(H // 8):
            xs_blk = xq_ref[sp, :, c * 8 + 8:c * 8 + 32, :]
            p = jnp.concatenate(
                [xs_blk[:, :, kw + 126:kw + 126 + WPAD].reshape(72, WPAD)
                 for kw in range(3)] + [ones_row],
                axis=0)
            r = mm(w_ref[:], p)
            r = jnp.maximum(r, 0.0)
            accs[c % 4] = accs[c % 4] + jnp.sum(r.reshape(8, IN_FEATS, WPAD),
                                                axis=0)
        acc = (accs[0] + accs[1]) + (accs[2] + accs[3])
        lane = jax.lax.broadcasted_iota(jnp.int32, (1, WPAD), 1)
        acc = jnp.where((lane >= 1) & (lane < W + 1), acc, 0.0)
        out_ref[0] = jnp.sum(acc, axis=1, keepdims=True) * (1.0 / (H * W))


def _gcn_head_kernel(h2_ref, atd_ref, abu_ref, mpool_ref, wtd_ref, btd_ref,
                     wbu_ref, bbu_ref, wg2_ref, bg2_ref, wfc_ref, bfc_ref,
                     out_ref):
    def mm(a, b):
        return jax.lax.dot_general(a, b, (((1,), (0,)), ((), ())),
                                   preferred_element_type=jnp.float32)

    h2 = h2_ref[:]
    td = jnp.maximum(mm(atd_ref[:], mm(h2, wtd_ref[:])) + btd_ref[:], 0.0)
    bu = jnp.maximum(mm(abu_ref[:], mm(h2, wbu_ref[:])) + bbu_ref[:], 0.0)
    z = jnp.concatenate([td, bu], axis=1)
    z2 = jnp.maximum(mm(atd_ref[:], mm(z, wg2_ref[:])) + bg2_ref[:], 0.0)
    out_ref[:] = mm(mm(mpool_ref[:], z2), wfc_ref[:]) + bfc_ref[:]


def kernel(x, W_conv, b_conv, W_td, b_td, W_bu, b_bu, W_g2, b_g2, W_fc, b_fc):
    shift = ((np.arange(24)[None, :, None] - 7 - np.arange(8)[:, None, None])
             == np.arange(3)[None, None, :]).astype(np.float32)
    w2 = jnp.einsum('oihw,drh->dowir', W_conv,
                    jnp.asarray(shift)).reshape(8 * IN_FEATS, 216)
    bc = jnp.tile(b_conv, 8).reshape(8 * IN_FEATS, 1)
    w2 = jnp.concatenate([w2, bc], axis=1).astype(jnp.bfloat16)

    full = lambda shape: pl.BlockSpec(shape, lambda i: tuple(0 for _ in shape))
    pooled = pl.pallas_call(
        _conv_pool_kernel,
        grid=(B + 1,),
        in_specs=[
            pl.BlockSpec((1, 3, H, W),
                         lambda i: (jnp.minimum(i, B - 1), 0, 0, 0)),
            full((8 * IN_FEATS, 217)),
        ],
        out_specs=pl.BlockSpec((1, IN_FEATS, 1),
                               lambda i: (jnp.maximum(i - 1, 0), 0, 0)),
        out_shape=jax.ShapeDtypeStruct((B, IN_FEATS, 1), jnp.float32),
        scratch_shapes=[pltpu.VMEM((2, 3, 248, 384), jnp.bfloat16)],
        compiler_params=pltpu.CompilerParams(
            dimension_semantics=("arbitrary",)),
    )(x, w2)

    h2 = pooled.reshape(B, NUM_NODES, D_NODE).transpose(1, 0, 2).reshape(
        NB, D_NODE)

    idx = np.arange(NUM_NODES)
    atd = np.where(idx[None, :] > idx[:, None],
                   1.0 / np.maximum(NUM_NODES - 1 - idx, 1)[:, None], 0.0)
    abu = np.where(idx[None, :] < idx[:, None],
                   1.0 / np.maximum(idx, 1)[:, None], 0.0)
    eye = np.eye(B, dtype=np.float32)
    atd_big = jnp.asarray(np.kron(atd, eye), jnp.float32)
    abu_big = jnp.asarray(np.kron(abu, eye), jnp.float32)
    mpool = jnp.asarray(np.kron(np.full((1, NUM_NODES), 1.0 / NUM_NODES,
                                        np.float32), eye), jnp.float32)

    num_classes = W_fc.shape[1]
    out = pl.pallas_call(
        _gcn_head_kernel,
        out_shape=jax.ShapeDtypeStruct((B, num_classes), jnp.float32),
    )(h2, atd_big, abu_big, mpool, W_td, b_td.reshape(1, HID), W_bu,
      b_bu.reshape(1, HID), W_g2, b_g2.reshape(1, HID), W_fc,
      b_fc.reshape(1, num_classes))
    return out


# speedup vs baseline: 1.1252x; 1.1252x over previous
import jax
import jax.numpy as jnp
import numpy as np
from jax.experimental import pallas as pl
from jax.experimental.pallas import tpu as pltpu

B = 64
IN_FEATS = 64
NUM_NODES = 16
D_NODE = 4
HID = 128
H = W = 224
WPAD = 256
HPAD = 232
NB = NUM_NODES * B


def _conv_pool_kernel(x_ref, w_ref, out_ref, xq_ref):
    def mm(a, b):
        return jax.lax.dot_general(a, b, (((1,), (0,)), ((), ())),
                                   preferred_element_type=jnp.float32)

    @pl.when(pl.program_id(0) == 0)
    def _zero():
        xq_ref[:] = jnp.zeros((2, 3, 248, 384), jnp.bfloat16)

    xq_ref[0, :, 16:240, 128:352] = x_ref[0].astype(jnp.bfloat16)
    xq_ref[1, :, 16:240, 128:352] = x_ref[1].astype(jnp.bfloat16)

    lane = jax.lax.broadcasted_iota(jnp.int32, (1, WPAD), 1)
    for s in range(2):
        accs = [jnp.zeros((IN_FEATS, WPAD), jnp.float32) for _ in range(4)]
        ones_row = jnp.ones((1, WPAD), jnp.bfloat16)
        for c in range(H // 8):
            xs_blk = xq_ref[s, :, c * 8 + 8:c * 8 + 32, :]
            p = jnp.concatenate(
                [xs_blk[:, :, kw + 126:kw + 126 + WPAD].reshape(72, WPAD)
                 for kw in range(3)] + [ones_row],
                axis=0)
            r = mm(w_ref[:], p)
            r = jnp.maximum(r, 0.0)
            accs[c % 4] = accs[c % 4] + jnp.sum(r.reshape(8, IN_FEATS, WPAD),
                                                axis=0)
        acc = (accs[0] + accs[1]) + (accs[2] + accs[3])
        acc = jnp.where((lane >= 1) & (lane < W + 1), acc, 0.0)
        out_ref[s] = jnp.sum(acc, axis=1, keepdims=True) * (1.0 / (H * W))


def _gcn_head_kernel(h2_ref, atd_ref, abu_ref, mpool_ref, wtd_ref, btd_ref,
                     wbu_ref, bbu_ref, wg2_ref, bg2_ref, wfc_ref, bfc_ref,
                     out_ref):
    def mm(a, b):
        return jax.lax.dot_general(a, b, (((1,), (0,)), ((), ())),
                                   preferred_element_type=jnp.float32)

    h2 = h2_ref[:]
    td = jnp.maximum(mm(atd_ref[:], mm(h2, wtd_ref[:])) + btd_ref[:], 0.0)
    bu = jnp.maximum(mm(abu_ref[:], mm(h2, wbu_ref[:])) + bbu_ref[:], 0.0)
    z = jnp.concatenate([td, bu], axis=1)
    z2 = jnp.maximum(mm(atd_ref[:], mm(z, wg2_ref[:])) + bg2_ref[:], 0.0)
    out_ref[:] = mm(mm(mpool_ref[:], z2), wfc_ref[:]) + bfc_ref[:]


def kernel(x, W_conv, b_conv, W_td, b_td, W_bu, b_bu, W_g2, b_g2, W_fc, b_fc):
    shift = ((np.arange(24)[None, :, None] - 7 - np.arange(8)[:, None, None])
             == np.arange(3)[None, None, :]).astype(np.float32)
    w2 = jnp.einsum('oihw,drh->dowir', W_conv,
                    jnp.asarray(shift)).reshape(8 * IN_FEATS, 216)
    bc = jnp.tile(b_conv, 8).reshape(8 * IN_FEATS, 1)
    w2 = jnp.concatenate([w2, bc], axis=1).astype(jnp.bfloat16)

    full = lambda shape: pl.BlockSpec(shape, lambda i: tuple(0 for _ in shape))
    pooled = pl.pallas_call(
        _conv_pool_kernel,
        grid=(B // 2,),
        in_specs=[
            pl.BlockSpec((2, 3, H, W), lambda i: (i, 0, 0, 0)),
            full((8 * IN_FEATS, 217)),
        ],
        out_specs=pl.BlockSpec((2, IN_FEATS, 1), lambda i: (i, 0, 0)),
        out_shape=jax.ShapeDtypeStruct((B, IN_FEATS, 1), jnp.float32),
        scratch_shapes=[pltpu.VMEM((2, 3, 248, 384), jnp.bfloat16)],
        compiler_params=pltpu.CompilerParams(
            dimension_semantics=("arbitrary",)),
    )(x, w2)

    h2 = pooled.reshape(B, NUM_NODES, D_NODE).transpose(1, 0, 2).reshape(
        NB, D_NODE)

    idx = np.arange(NUM_NODES)
    atd = np.where(idx[None, :] > idx[:, None],
                   1.0 / np.maximum(NUM_NODES - 1 - idx, 1)[:, None], 0.0)
    abu = np.where(idx[None, :] < idx[:, None],
                   1.0 / np.maximum(idx, 1)[:, None], 0.0)
    eye = np.eye(B, dtype=np.float32)
    atd_big = jnp.asarray(np.kron(atd, eye), jnp.float32)
    abu_big = jnp.asarray(np.kron(abu, eye), jnp.float32)
    mpool = jnp.asarray(np.kron(np.full((1, NUM_NODES), 1.0 / NUM_NODES,
                                        np.float32), eye), jnp.float32)

    num_classes = W_fc.shape[1]
    out = pl.pallas_call(
        _gcn_head_kernel,
        out_shape=jax.ShapeDtypeStruct((B, num_classes), jnp.float32),
    )(h2, atd_big, abu_big, mpool, W_td, b_td.reshape(1, HID), W_bu,
      b_bu.reshape(1, HID), W_g2, b_g2.reshape(1, HID), W_fc,
      b_fc.reshape(1, num_classes))
    return out
